# Initial kernel scaffold; baseline (speedup 1.0000x reference)
import jax
import jax.numpy as jnp
from jax.experimental import pallas as pl


def kernel(scores, bbox_deltas, im_info, cfg_key):
    raise NotImplementedError("write your pallas kernel here")


# speedup vs baseline: 1.3472x; 1.3472x over previous
import functools

import numpy as np
import jax
import jax.numpy as jnp
from jax import lax
from jax.experimental import pallas as pl
from jax.experimental.pallas import tpu as pltpu

_FEAT_STRIDE = 16
_PRE_N = 6000
_POST_N = 300
_IOU_T = 0.7
_B, _H, _W, _A = 8, 50, 76, 9
_N = _H * _W * _A

_INTERPRET = False


def _anchor_params():
    ratios = np.array([0.5, 1.0, 2.0])
    scales = np.array([8.0, 16.0, 32.0])

    def whctrs(a):
        w = a[2] - a[0] + 1.0
        h = a[3] - a[1] + 1.0
        return w, h, a[0] + 0.5 * (w - 1.0), a[1] + 0.5 * (h - 1.0)

    def mk(ws, hs, xc, yc):
        ws = ws[:, None]
        hs = hs[:, None]
        return np.hstack([
            xc - 0.5 * (ws - 1.0), yc - 0.5 * (hs - 1.0),
            xc + 0.5 * (ws - 1.0), yc + 0.5 * (hs - 1.0),
        ])

    base = np.array([0.0, 0.0, 15.0, 15.0])
    w, h, xc, yc = whctrs(base)
    size_ratios = (w * h) / ratios
    ws = np.round(np.sqrt(size_ratios))
    hs = np.round(ws * ratios)
    ra = mk(ws, hs, xc, yc)
    rows = []
    for i in range(ra.shape[0]):
        wi, hi, xci, yci = whctrs(ra[i])
        rows.append(mk(wi * scales, hi * scales, xci, yci))
    anchors32 = np.vstack(rows).astype(np.float32)

    shift_x = np.arange(_W, dtype=np.float32) * np.float32(_FEAT_STRIDE)
    shift_y = np.arange(_H, dtype=np.float32) * np.float32(_FEAT_STRIDE)
    sx, sy = np.meshgrid(shift_x, shift_y)
    shifts = np.stack([sx.ravel(), sy.ravel(), sx.ravel(), sy.ravel()],
                      axis=1).astype(np.float32)
    full = (anchors32[None, :, :] + shifts[:, None, :]).reshape(-1, 4)
    aw = (full[:, 2] - full[:, 0]) + np.float32(1.0)
    ah = (full[:, 3] - full[:, 1]) + np.float32(1.0)
    acx = full[:, 0] + np.float32(0.5) * aw
    acy = full[:, 1] + np.float32(0.5) * ah
    return (aw.reshape(1, _N), ah.reshape(1, _N),
            acx.reshape(1, _N), acy.reshape(1, _N))


_AW, _AH, _ACX, _ACY = _anchor_params()


def _nms_body(sc_ref, dx_ref, dy_ref, dw_ref, dh_ref,
              aw_ref, ah_ref, acx_ref, acy_ref, im_ref,
              ox1_ref, oy1_ref, ox2_ref, oy2_ref,
              x1_s, y1_s, x2_s, y2_s, ar_s, scw_s, key_s):
    NEG = jnp.float32(-jnp.inf)
    aw = aw_ref[...]
    ah = ah_ref[...]
    acx = acx_ref[...]
    acy = acy_ref[...]

    pcx = dx_ref[...] * aw + acx
    pcy = dy_ref[...] * ah + acy
    pw = jnp.exp(dw_ref[...]) * aw
    ph = jnp.exp(dh_ref[...]) * ah
    x1 = pcx - 0.5 * pw
    y1 = pcy - 0.5 * ph
    x2 = pcx + 0.5 * pw
    y2 = pcy + 0.5 * ph
    xmax = im_ref[:, 1:2] - 1.0
    ymax = im_ref[:, 0:1] - 1.0
    x1 = jnp.minimum(jnp.maximum(x1, 0.0), xmax)
    x2 = jnp.minimum(jnp.maximum(x2, 0.0), xmax)
    y1 = jnp.minimum(jnp.maximum(y1, 0.0), ymax)
    y2 = jnp.minimum(jnp.maximum(y2, 0.0), ymax)
    x1_s[...] = x1
    y1_s[...] = y1
    x2_s[...] = x2
    y2_s[...] = y2
    ar_s[...] = (x2 - x1 + 1.0) * (y2 - y1 + 1.0)

    sc = sc_ref[...]
    u = lax.bitcast_convert_type(sc, jnp.int32)
    imin = jnp.int32(-2147483648)
    key_s[...] = jnp.where(u >= 0, u, imin - u)

    def bs_body(_, lohi):
        lo, hi = lohi
        mid = (lo >> 1) + (hi >> 1) + (lo & hi & 1)
        cnt = jnp.sum((key_s[...] >= mid).astype(jnp.int32), axis=1,
                      keepdims=True)
        ge = cnt >= _PRE_N
        return jnp.where(ge, mid, lo), jnp.where(ge, hi, mid)

    lo0 = jnp.full((_B, 1), imin, jnp.int32)
    hi0 = jnp.full((_B, 1), 2147483647, jnp.int32)
    tkey, _ = lax.fori_loop(0, 32, bs_body, (lo0, hi0))

    scw_s[...] = jnp.where(key_s[...] >= tkey, sc, NEG)

    iota = lax.broadcasted_iota(jnp.int32, (_B, _N), 1)

    def pick(scw):
        m = jnp.max(scw, axis=1, keepdims=True)
        idx = jnp.min(jnp.where(scw == m, iota, _N), axis=1, keepdims=True)
        oh = iota == idx

        def g(ref):
            return jnp.sum(jnp.where(oh, ref[...], 0.0), axis=1, keepdims=True)

        return m, oh, g(x1_s), g(y1_s), g(x2_s), g(y2_s), g(ar_s)

    _, _, bx1, by1, bx2, by2, _ = pick(scw_s[...])

    lane_out = lax.broadcasted_iota(jnp.int32, (_B, _POST_N), 1)

    def body(t, _):
        scw = scw_s[...]
        m, oh, sx1, sy1, sx2, sy2, sar = pick(scw)
        ex = m == NEG
        ux1 = jnp.where(ex, bx1, sx1)
        uy1 = jnp.where(ex, by1, sy1)
        ux2 = jnp.where(ex, bx2, sx2)
        uy2 = jnp.where(ex, by2, sy2)
        sel = lane_out == t
        ox1_ref[...] = jnp.where(sel, ux1, ox1_ref[...])
        oy1_ref[...] = jnp.where(sel, uy1, oy1_ref[...])
        ox2_ref[...] = jnp.where(sel, ux2, ox2_ref[...])
        oy2_ref[...] = jnp.where(sel, uy2, oy2_ref[...])
        xx1 = jnp.maximum(sx1, x1_s[...])
        yy1 = jnp.maximum(sy1, y1_s[...])
        xx2 = jnp.minimum(sx2, x2_s[...])
        yy2 = jnp.minimum(sy2, y2_s[...])
        w = jnp.maximum(0.0, xx2 - xx1 + 1.0)
        h = jnp.maximum(0.0, yy2 - yy1 + 1.0)
        inter = w * h
        iou = inter / (sar + ar_s[...] - inter)
        supp = (iou > _IOU_T) | oh
        scw_s[...] = jnp.where(supp, NEG, scw)
        return 0

    lax.fori_loop(0, _POST_N, body, 0)


def kernel(scores, bbox_deltas, im_info, cfg_key):
    sc = scores[:, _A:, :, :].transpose(0, 2, 3, 1).reshape(_B, _N)
    d = bbox_deltas.transpose(0, 2, 3, 1).reshape(_B, _N, 4)
    dx, dy, dw, dh = d[..., 0], d[..., 1], d[..., 2], d[..., 3]

    outs = pl.pallas_call(
        _nms_body,
        out_shape=[jax.ShapeDtypeStruct((_B, _POST_N), jnp.float32)] * 4,
        scratch_shapes=[pltpu.VMEM((_B, _N), jnp.float32)] * 6
        + [pltpu.VMEM((_B, _N), jnp.int32)],
        interpret=_INTERPRET,
    )(sc, dx, dy, dw, dh,
      jnp.asarray(_AW), jnp.asarray(_AH), jnp.asarray(_ACX), jnp.asarray(_ACY),
      im_info)
    x1, y1, x2, y2 = outs
    boxes = jnp.stack([x1, y1, x2, y2], axis=-1)
    bid = jnp.broadcast_to(
        jnp.arange(_B, dtype=jnp.float32)[:, None, None], (_B, _POST_N, 1))
    return jnp.concatenate([bid, boxes], axis=2)


# speedup vs baseline: 2.2727x; 1.6870x over previous
import functools

import numpy as np
import jax
import jax.numpy as jnp
from jax import lax
from jax.experimental import pallas as pl
from jax.experimental.pallas import tpu as pltpu
from jax.experimental.pallas import tpu_sc as plsc

_FEAT_STRIDE = 16
_PRE_N = 6000
_POST_N = 300
_IOU_T = 0.7
_B, _H, _W, _A = 8, 50, 76, 9
_N = _H * _W * _A
_CHUNK = 4288
_NCHUNKS = 8
_NPAD = _CHUNK * _NCHUNKS
_CAP = 6144

_INTERPRET = False


def _anchor_params():
    ratios = np.array([0.5, 1.0, 2.0])
    scales = np.array([8.0, 16.0, 32.0])

    def whctrs(a):
        w = a[2] - a[0] + 1.0
        h = a[3] - a[1] + 1.0
        return w, h, a[0] + 0.5 * (w - 1.0), a[1] + 0.5 * (h - 1.0)

    def mk(ws, hs, xc, yc):
        ws = ws[:, None]
        hs = hs[:, None]
        return np.hstack([
            xc - 0.5 * (ws - 1.0), yc - 0.5 * (hs - 1.0),
            xc + 0.5 * (ws - 1.0), yc + 0.5 * (hs - 1.0),
        ])

    base = np.array([0.0, 0.0, 15.0, 15.0])
    w, h, xc, yc = whctrs(base)
    size_ratios = (w * h) / ratios
    ws = np.round(np.sqrt(size_ratios))
    hs = np.round(ws * ratios)
    ra = mk(ws, hs, xc, yc)
    rows = []
    for i in range(ra.shape[0]):
        wi, hi, xci, yci = whctrs(ra[i])
        rows.append(mk(wi * scales, hi * scales, xci, yci))
    anchors32 = np.vstack(rows).astype(np.float32)

    shift_x = np.arange(_W, dtype=np.float32) * np.float32(_FEAT_STRIDE)
    shift_y = np.arange(_H, dtype=np.float32) * np.float32(_FEAT_STRIDE)
    sx, sy = np.meshgrid(shift_x, shift_y)
    shifts = np.stack([sx.ravel(), sy.ravel(), sx.ravel(), sy.ravel()],
                      axis=1).astype(np.float32)
    full = (anchors32[None, :, :] + shifts[:, None, :]).reshape(-1, 4)
    aw = (full[:, 2] - full[:, 0]) + np.float32(1.0)
    ah = (full[:, 3] - full[:, 1]) + np.float32(1.0)
    acx = full[:, 0] + np.float32(0.5) * aw
    acy = full[:, 1] + np.float32(0.5) * ah
    return (aw.reshape(1, _N), ah.reshape(1, _N),
            acx.reshape(1, _N), acy.reshape(1, _N))


_AW, _AH, _ACX, _ACY = _anchor_params()


def _tc1_body(sc_ref, dx_ref, dy_ref, dw_ref, dh_ref,
              aw_ref, ah_ref, acx_ref, acy_ref, im_ref,
              ox1_ref, oy1_ref, ox2_ref, oy2_ref, oar_ref, osw_ref,
              key_s):
    NEG = jnp.float32(-jnp.inf)
    aw = aw_ref[...]
    ah = ah_ref[...]

    pcx = dx_ref[...] * aw + acx_ref[...]
    pcy = dy_ref[...] * ah + acy_ref[...]
    pw = jnp.exp(dw_ref[...]) * aw
    ph = jnp.exp(dh_ref[...]) * ah
    x1 = pcx - 0.5 * pw
    y1 = pcy - 0.5 * ph
    x2 = pcx + 0.5 * pw
    y2 = pcy + 0.5 * ph
    xmax = im_ref[:, 1:2] - 1.0
    ymax = im_ref[:, 0:1] - 1.0
    x1 = jnp.minimum(jnp.maximum(x1, 0.0), xmax)
    x2 = jnp.minimum(jnp.maximum(x2, 0.0), xmax)
    y1 = jnp.minimum(jnp.maximum(y1, 0.0), ymax)
    y2 = jnp.minimum(jnp.maximum(y2, 0.0), ymax)

    sc = sc_ref[...]
    u = lax.bitcast_convert_type(sc, jnp.int32)
    imin = jnp.int32(-2147483648)
    key_s[...] = jnp.where(u >= 0, u, imin - u)

    def bs_body(_, lohi):
        lo, hi = lohi
        mid = (lo >> 1) + (hi >> 1) + (lo & hi & 1)
        cnt = jnp.sum((key_s[...] >= mid).astype(jnp.int32), axis=1,
                      keepdims=True)
        ge = cnt >= _PRE_N
        return jnp.where(ge, mid, lo), jnp.where(ge, hi, mid)

    lo0 = jnp.full((_B, 1), imin, jnp.int32)
    hi0 = jnp.full((_B, 1), 2147483647, jnp.int32)
    tkey, _ = lax.fori_loop(0, 32, bs_body, (lo0, hi0))
    scw = jnp.where(key_s[...] >= tkey, sc, NEG)

    zero_tail = jnp.zeros((_B, _NPAD - _N), jnp.float32)
    ox1_ref[:, :_N] = x1
    oy1_ref[:, :_N] = y1
    ox2_ref[:, :_N] = x2
    oy2_ref[:, :_N] = y2
    oar_ref[:, :_N] = (x2 - x1 + 1.0) * (y2 - y1 + 1.0)
    osw_ref[:, :_N] = scw
    ox1_ref[:, _N:] = zero_tail
    oy1_ref[:, _N:] = zero_tail
    ox2_ref[:, _N:] = zero_tail
    oy2_ref[:, _N:] = zero_tail
    oar_ref[:, _N:] = zero_tail
    osw_ref[:, _N:] = zero_tail + NEG


def _sc_body(*refs):
    hin = refs[0:6]
    hout = refs[6:12]
    cin = refs[12:18]
    cout = refs[18:24]
    c = lax.axis_index("c")
    s = lax.axis_index("s")
    img = s * 2 + c
    NEG = jnp.float32(-jnp.inf)

    @pl.when(img < _B)
    def _():
        def init_body(i, carry):
            sl = pl.ds(i * 16, 16)
            for r in cout[:5]:
                r[sl] = jnp.zeros((16,), jnp.float32)
            cout[5][sl] = jnp.full((16,), NEG, jnp.float32)
            return carry
        lax.fori_loop(0, _CAP // 16, init_body, 0)

        def chunk_body(ci, wptr):
            base = img * _NPAD + ci * _CHUNK
            for hr, vr in zip(hin, cin):
                pltpu.sync_copy(hr.at[pl.ds(base, _CHUNK)], vr)

            def vec_body(i, wp):
                sl = pl.ds(i * 16, 16)
                mask = cin[5][sl] > NEG
                cnt = jnp.sum(mask.astype(jnp.int32))

                @pl.when(wp <= _CAP - 16)
                def _():
                    for vr, orr in zip(cin, cout):
                        plsc.store_compressed(orr.at[pl.ds(wp, 16)],
                                              vr[sl], mask=mask)
                return wp + cnt

            return lax.fori_loop(0, _CHUNK // 16, vec_body, wptr)

        lax.fori_loop(0, _NCHUNKS, chunk_body, jnp.int32(0))
        for orr, hr in zip(cout, hout):
            pltpu.sync_copy(orr, hr.at[pl.ds(img * _CAP, _CAP)])


@functools.lru_cache(maxsize=1)
def _get_sc_compact():
    return pl.kernel(
        _sc_body,
        out_type=[jax.ShapeDtypeStruct((_B * _CAP,), jnp.float32)] * 6,
        mesh=plsc.VectorSubcoreMesh(core_axis_name="c", subcore_axis_name="s"),
        scratch_types=[pltpu.VMEM((_CHUNK,), jnp.float32)] * 6
        + [pltpu.VMEM((_CAP,), jnp.float32)] * 6,
        compiler_params=pltpu.CompilerParams(needs_layout_passes=False),
    )


def _tc2_body(x1_ref, y1_ref, x2_ref, y2_ref, ar_ref, sw_ref,
              ox1_ref, oy1_ref, ox2_ref, oy2_ref, scw_s):
    NEG = jnp.float32(-jnp.inf)
    scw_s[...] = sw_ref[...]
    iota = lax.broadcasted_iota(jnp.int32, (_B, _CAP), 1)

    def pick(scw):
        m = jnp.max(scw, axis=1, keepdims=True)
        idx = jnp.min(jnp.where(scw == m, iota, _CAP), axis=1, keepdims=True)
        oh = iota == idx

        def g(ref):
            return jnp.sum(jnp.where(oh, ref[...], 0.0), axis=1, keepdims=True)

        return m, oh, g(x1_ref), g(y1_ref), g(x2_ref), g(y2_ref)

    _, _, bx1, by1, bx2, by2 = pick(scw_s[...])

    lane_out = lax.broadcasted_iota(jnp.int32, (_B, _POST_N), 1)

    def body(t, _):
        scw = scw_s[...]
        m, oh, sx1, sy1, sx2, sy2 = pick(scw)
        sar = (sx2 - sx1 + 1.0) * (sy2 - sy1 + 1.0)
        ex = m == NEG
        sel = lane_out == t
        ox1_ref[...] = jnp.where(sel, jnp.where(ex, bx1, sx1), ox1_ref[...])
        oy1_ref[...] = jnp.where(sel, jnp.where(ex, by1, sy1), oy1_ref[...])
        ox2_ref[...] = jnp.where(sel, jnp.where(ex, bx2, sx2), ox2_ref[...])
        oy2_ref[...] = jnp.where(sel, jnp.where(ex, by2, sy2), oy2_ref[...])
        xx1 = jnp.maximum(sx1, x1_ref[...])
        yy1 = jnp.maximum(sy1, y1_ref[...])
        xx2 = jnp.minimum(sx2, x2_ref[...])
        yy2 = jnp.minimum(sy2, y2_ref[...])
        w = jnp.maximum(0.0, xx2 - xx1 + 1.0)
        h = jnp.maximum(0.0, yy2 - yy1 + 1.0)
        inter = w * h
        iou = inter / (sar + ar_ref[...] - inter)
        supp = (iou > _IOU_T) | oh
        scw_s[...] = jnp.where(supp, NEG, scw)
        return 0

    lax.fori_loop(0, _POST_N, body, 0)


def kernel(scores, bbox_deltas, im_info, cfg_key):
    sc = scores[:, _A:, :, :].transpose(0, 2, 3, 1).reshape(_B, _N)
    d = bbox_deltas.transpose(0, 2, 3, 1).reshape(_B, _N, 4)
    dx, dy, dw, dh = d[..., 0], d[..., 1], d[..., 2], d[..., 3]

    staged = pl.pallas_call(
        _tc1_body,
        out_shape=[jax.ShapeDtypeStruct((_B, _NPAD), jnp.float32)] * 6,
        scratch_shapes=[pltpu.VMEM((_B, _N), jnp.int32)],
        interpret=_INTERPRET,
    )(sc, dx, dy, dw, dh,
      jnp.asarray(_AW), jnp.asarray(_AH), jnp.asarray(_ACX), jnp.asarray(_ACY),
      im_info)

    compacted = _get_sc_compact()(*[a.reshape(_B * _NPAD) for a in staged])
    compacted = [a.reshape(_B, _CAP) for a in compacted]

    outs = pl.pallas_call(
        _tc2_body,
        out_shape=[jax.ShapeDtypeStruct((_B, _POST_N), jnp.float32)] * 4,
        scratch_shapes=[pltpu.VMEM((_B, _CAP), jnp.float32)],
        interpret=_INTERPRET,
    )(*compacted)

    x1, y1, x2, y2 = outs
    boxes = jnp.stack([x1, y1, x2, y2], axis=-1)
    bid = jnp.broadcast_to(
        jnp.arange(_B, dtype=jnp.float32)[:, None, None], (_B, _POST_N, 1))
    return jnp.concatenate([bid, boxes], axis=2)


# speedup vs baseline: 4.5913x; 2.0203x over previous
import functools

import numpy as np
import jax
import jax.numpy as jnp
from jax import lax
from jax.experimental import pallas as pl
from jax.experimental.pallas import tpu as pltpu
from jax.experimental.pallas import tpu_sc as plsc

_FEAT_STRIDE = 16
_PRE_N = 6000
_POST_N = 300
_IOU_T = 0.7
_B, _H, _W, _A = 8, 50, 76, 9
_HW = _H * _W
_HWP = 3840
_N = _HW * _A
_NPAD = _HWP * _A
_CHUNK = _NPAD // 8
_NCHUNKS = 8
_CAP = 6144
_BIGIDX = 1 << 28

_INTERPRET = False


def _anchor_params():
    ratios = np.array([0.5, 1.0, 2.0])
    scales = np.array([8.0, 16.0, 32.0])

    def whctrs(a):
        w = a[2] - a[0] + 1.0
        h = a[3] - a[1] + 1.0
        return w, h, a[0] + 0.5 * (w - 1.0), a[1] + 0.5 * (h - 1.0)

    def mk(ws, hs, xc, yc):
        ws = ws[:, None]
        hs = hs[:, None]
        return np.hstack([
            xc - 0.5 * (ws - 1.0), yc - 0.5 * (hs - 1.0),
            xc + 0.5 * (ws - 1.0), yc + 0.5 * (hs - 1.0),
        ])

    base = np.array([0.0, 0.0, 15.0, 15.0])
    w, h, xc, yc = whctrs(base)
    size_ratios = (w * h) / ratios
    ws = np.round(np.sqrt(size_ratios))
    hs = np.round(ws * ratios)
    ra = mk(ws, hs, xc, yc)
    rows = []
    for i in range(ra.shape[0]):
        wi, hi, xci, yci = whctrs(ra[i])
        rows.append(mk(wi * scales, hi * scales, xci, yci))
    anchors32 = np.vstack(rows).astype(np.float32)

    shift_x = np.arange(_W, dtype=np.float32) * np.float32(_FEAT_STRIDE)
    shift_y = np.arange(_H, dtype=np.float32) * np.float32(_FEAT_STRIDE)
    sx, sy = np.meshgrid(shift_x, shift_y)
    shifts = np.stack([sx.ravel(), sy.ravel(), sx.ravel(), sy.ravel()],
                      axis=1).astype(np.float32)
    full = (anchors32[None, :, :] + shifts[:, None, :]).reshape(-1, 4)
    aw = (full[:, 2] - full[:, 0]) + np.float32(1.0)
    ah = (full[:, 3] - full[:, 1]) + np.float32(1.0)
    acx = full[:, 0] + np.float32(0.5) * aw
    acy = full[:, 1] + np.float32(0.5) * ah

    def raw(v):
        return np.ascontiguousarray(v.reshape(_HW, _A).T).reshape(1, _A, _HW)

    return raw(aw), raw(ah), raw(acx), raw(acy)


_AW, _AH, _ACX, _ACY = _anchor_params()


def _tc1_body(sc_ref, d_ref, aw_ref, ah_ref, acx_ref, acy_ref, im_ref,
              ox1_ref, oy1_ref, ox2_ref, oy2_ref, oar_ref, osw_ref):
    NEG = jnp.float32(-jnp.inf)
    aw = aw_ref[...]
    ah = ah_ref[...]

    pcx = d_ref[:, :, 0, :] * aw + acx_ref[...]
    pcy = d_ref[:, :, 1, :] * ah + acy_ref[...]
    pw = jnp.exp(d_ref[:, :, 2, :]) * aw
    ph = jnp.exp(d_ref[:, :, 3, :]) * ah
    x1 = pcx - 0.5 * pw
    y1 = pcy - 0.5 * ph
    x2 = pcx + 0.5 * pw
    y2 = pcy + 0.5 * ph
    xmax = (im_ref[:, 1:2] - 1.0)[:, :, None]
    ymax = (im_ref[:, 0:1] - 1.0)[:, :, None]
    x1 = jnp.minimum(jnp.maximum(x1, 0.0), xmax)
    x2 = jnp.minimum(jnp.maximum(x2, 0.0), xmax)
    y1 = jnp.minimum(jnp.maximum(y1, 0.0), ymax)
    y2 = jnp.minimum(jnp.maximum(y2, 0.0), ymax)

    sc = sc_ref[:, _A:, :]
    u = lax.bitcast_convert_type(sc, jnp.int32)
    imin = jnp.int32(-2147483648)
    keys = jnp.where(u >= 0, u, imin - u)

    def bs_body(_, lohi):
        lo, hi = lohi
        mid = (lo >> 1) + (hi >> 1) + (lo & hi & 1)
        cnt = jnp.sum(
            jnp.sum((keys >= mid).astype(jnp.int32), axis=2, keepdims=True),
            axis=1, keepdims=True)
        ge = cnt >= _PRE_N
        return jnp.where(ge, mid, lo), jnp.where(ge, hi, mid)

    lo0 = jnp.full((_B, 1, 1), imin, jnp.int32)
    hi0 = jnp.full((_B, 1, 1), 2147483647, jnp.int32)
    tkey, _ = lax.fori_loop(0, 32, bs_body, (lo0, hi0))
    scw = jnp.where(keys >= tkey, sc, NEG)

    fill = jnp.zeros((_B, _A, _HWP - _HW), jnp.float32)
    ox1_ref[:, :, :_HW] = x1
    oy1_ref[:, :, :_HW] = y1
    ox2_ref[:, :, :_HW] = x2
    oy2_ref[:, :, :_HW] = y2
    oar_ref[:, :, :_HW] = (x2 - x1 + 1.0) * (y2 - y1 + 1.0)
    osw_ref[:, :, :_HW] = scw
    ox1_ref[:, :, _HW:] = fill
    oy1_ref[:, :, _HW:] = fill
    ox2_ref[:, :, _HW:] = fill
    oy2_ref[:, :, _HW:] = fill
    oar_ref[:, :, _HW:] = fill
    osw_ref[:, :, _HW:] = fill + NEG


def _sc_body(*refs):
    hin = refs[0:6]
    hout = refs[6:13]
    cin = refs[13:19]
    cout = refs[19:25]
    coi = refs[25]
    c = lax.axis_index("c")
    s = lax.axis_index("s")
    img = s * 2 + c
    NEG = jnp.float32(-jnp.inf)
    lane = lax.iota(jnp.int32, 16)

    @pl.when(img < _B)
    def _():
        def init_body(i, carry):
            sl = pl.ds(i * 16, 16)
            for r in cout[:5]:
                r[sl] = jnp.zeros((16,), jnp.float32)
            cout[5][sl] = jnp.full((16,), NEG, jnp.float32)
            coi[sl] = jnp.zeros((16,), jnp.int32)
            return carry
        lax.fori_loop(0, _CAP // 16, init_body, 0)

        def chunk_body(ci, wptr):
            base = img * _NPAD + ci * _CHUNK
            for hr, vr in zip(hin, cin):
                pltpu.sync_copy(hr.at[pl.ds(base, _CHUNK)], vr)

            def vec_body(i, wp):
                sl = pl.ds(i * 16, 16)
                mask = cin[5][sl] > NEG
                cnt = jnp.sum(mask.astype(jnp.int32))
                p = ci * _CHUNK + i * 16 + lane
                a = p // _HWP
                hw = p - a * _HWP
                oi = hw * _A + a

                @pl.when(wp <= _CAP - 16)
                def _():
                    for vr, orr in zip(cin, cout):
                        plsc.store_compressed(orr.at[pl.ds(wp, 16)],
                                              vr[sl], mask=mask)
                    plsc.store_compressed(coi.at[pl.ds(wp, 16)], oi, mask=mask)
                return wp + cnt

            return lax.fori_loop(0, _CHUNK // 16, vec_body, wptr)

        lax.fori_loop(0, _NCHUNKS, chunk_body, jnp.int32(0))
        for orr, hr in zip(cout, hout[:6]):
            pltpu.sync_copy(orr, hr.at[pl.ds(img * _CAP, _CAP)])
        pltpu.sync_copy(coi, hout[6].at[pl.ds(img * _CAP, _CAP)])


@functools.lru_cache(maxsize=1)
def _get_sc_compact():
    return pl.kernel(
        _sc_body,
        out_type=[jax.ShapeDtypeStruct((_B * _CAP,), jnp.float32)] * 6
        + [jax.ShapeDtypeStruct((_B * _CAP,), jnp.int32)],
        mesh=plsc.VectorSubcoreMesh(core_axis_name="c", subcore_axis_name="s"),
        scratch_types=[pltpu.VMEM((_CHUNK,), jnp.float32)] * 6
        + [pltpu.VMEM((_CAP,), jnp.float32)] * 6
        + [pltpu.VMEM((_CAP,), jnp.int32)],
        compiler_params=pltpu.CompilerParams(needs_layout_passes=False),
    )


def _tc2_body(x1_ref, y1_ref, x2_ref, y2_ref, ar_ref, sw_ref, oi_ref,
              ox1_ref, oy1_ref, ox2_ref, oy2_ref, scw_s):
    NEG = jnp.float32(-jnp.inf)
    scw_s[...] = sw_ref[...]
    oi = oi_ref[...]

    def pick(scw):
        m = jnp.max(scw, axis=1, keepdims=True)
        pidx = jnp.min(jnp.where(scw == m, oi, _BIGIDX), axis=1,
                       keepdims=True)
        oh = oi == pidx

        def g(ref):
            return jnp.sum(jnp.where(oh, ref[...], 0.0), axis=1, keepdims=True)

        return m, oh, g(x1_ref), g(y1_ref), g(x2_ref), g(y2_ref)

    _, _, bx1, by1, bx2, by2 = pick(scw_s[...])

    lane_out = lax.broadcasted_iota(jnp.int32, (_B, _POST_N), 1)

    def body(t, _):
        scw = scw_s[...]
        m, oh, sx1, sy1, sx2, sy2 = pick(scw)
        sar = (sx2 - sx1 + 1.0) * (sy2 - sy1 + 1.0)
        ex = m == NEG
        sel = lane_out == t
        ox1_ref[...] = jnp.where(sel, jnp.where(ex, bx1, sx1), ox1_ref[...])
        oy1_ref[...] = jnp.where(sel, jnp.where(ex, by1, sy1), oy1_ref[...])
        ox2_ref[...] = jnp.where(sel, jnp.where(ex, bx2, sx2), ox2_ref[...])
        oy2_ref[...] = jnp.where(sel, jnp.where(ex, by2, sy2), oy2_ref[...])
        xx1 = jnp.maximum(sx1, x1_ref[...])
        yy1 = jnp.maximum(sy1, y1_ref[...])
        xx2 = jnp.minimum(sx2, x2_ref[...])
        yy2 = jnp.minimum(sy2, y2_ref[...])
        w = jnp.maximum(0.0, xx2 - xx1 + 1.0)
        h = jnp.maximum(0.0, yy2 - yy1 + 1.0)
        inter = w * h
        iou = inter / (sar + ar_ref[...] - inter)
        supp = (iou > _IOU_T) | oh
        scw_s[...] = jnp.where(supp, NEG, scw)
        return 0

    lax.fori_loop(0, _POST_N, body, 0)


def kernel(scores, bbox_deltas, im_info, cfg_key):
    sc_raw = scores.reshape(_B, 2 * _A, _HW)
    d_raw = bbox_deltas.reshape(_B, _A, 4, _HW)

    staged = pl.pallas_call(
        _tc1_body,
        out_shape=[jax.ShapeDtypeStruct((_B, _A, _HWP), jnp.float32)] * 6,
        interpret=_INTERPRET,
    )(sc_raw, d_raw,
      jnp.asarray(_AW), jnp.asarray(_AH), jnp.asarray(_ACX), jnp.asarray(_ACY),
      im_info)

    compacted = _get_sc_compact()(*[a.reshape(_B * _NPAD) for a in staged])
    compacted = [a.reshape(_B, _CAP) for a in compacted]

    outs = pl.pallas_call(
        _tc2_body,
        out_shape=[jax.ShapeDtypeStruct((_B, _POST_N), jnp.float32)] * 4,
        scratch_shapes=[pltpu.VMEM((_B, _CAP), jnp.float32)],
        interpret=_INTERPRET,
    )(*compacted)

    x1, y1, x2, y2 = outs
    boxes = jnp.stack([x1, y1, x2, y2], axis=-1)
    bid = jnp.broadcast_to(
        jnp.arange(_B, dtype=jnp.float32)[:, None, None], (_B, _POST_N, 1))
    return jnp.concatenate([bid, boxes], axis=2)


# speedup vs baseline: 5.1182x; 1.1147x over previous
import functools

import numpy as np
import jax
import jax.numpy as jnp
from jax import lax
from jax.experimental import pallas as pl
from jax.experimental.pallas import tpu as pltpu
from jax.experimental.pallas import tpu_sc as plsc

_FEAT_STRIDE = 16
_PRE_N = 6000
_POST_N = 300
_IOU_T = 0.7
_B, _H, _W, _A = 8, 50, 76, 9
_HW = _H * _W
_HWP = 3840
_N = _HW * _A
_NPAD = _HWP * _A
_CHUNK = _NPAD // 8
_NCHUNKS = 8
_CAP = 6144
_BIGIDX = 1 << 28

_INTERPRET = False


def _anchor_params():
    ratios = np.array([0.5, 1.0, 2.0])
    scales = np.array([8.0, 16.0, 32.0])

    def whctrs(a):
        w = a[2] - a[0] + 1.0
        h = a[3] - a[1] + 1.0
        return w, h, a[0] + 0.5 * (w - 1.0), a[1] + 0.5 * (h - 1.0)

    def mk(ws, hs, xc, yc):
        ws = ws[:, None]
        hs = hs[:, None]
        return np.hstack([
            xc - 0.5 * (ws - 1.0), yc - 0.5 * (hs - 1.0),
            xc + 0.5 * (ws - 1.0), yc + 0.5 * (hs - 1.0),
        ])

    base = np.array([0.0, 0.0, 15.0, 15.0])
    w, h, xc, yc = whctrs(base)
    size_ratios = (w * h) / ratios
    ws = np.round(np.sqrt(size_ratios))
    hs = np.round(ws * ratios)
    ra = mk(ws, hs, xc, yc)
    rows = []
    for i in range(ra.shape[0]):
        wi, hi, xci, yci = whctrs(ra[i])
        rows.append(mk(wi * scales, hi * scales, xci, yci))
    anchors32 = np.vstack(rows).astype(np.float32)

    shift_x = np.arange(_W, dtype=np.float32) * np.float32(_FEAT_STRIDE)
    shift_y = np.arange(_H, dtype=np.float32) * np.float32(_FEAT_STRIDE)
    sx, sy = np.meshgrid(shift_x, shift_y)
    shifts = np.stack([sx.ravel(), sy.ravel(), sx.ravel(), sy.ravel()],
                      axis=1).astype(np.float32)
    full = (anchors32[None, :, :] + shifts[:, None, :]).reshape(-1, 4)
    aw = (full[:, 2] - full[:, 0]) + np.float32(1.0)
    ah = (full[:, 3] - full[:, 1]) + np.float32(1.0)
    acx = full[:, 0] + np.float32(0.5) * aw
    acy = full[:, 1] + np.float32(0.5) * ah

    def raw(v):
        return np.ascontiguousarray(v.reshape(_HW, _A).T).reshape(1, _A, _HW)

    return raw(aw), raw(ah), raw(acx), raw(acy)


_AW, _AH, _ACX, _ACY = _anchor_params()


def _tc1_body(sc_ref, d_ref, aw_ref, ah_ref, acx_ref, acy_ref, im_ref,
              ox1_ref, oy1_ref, ox2_ref, oy2_ref, oar_ref, osw_ref):
    NEG = jnp.float32(-jnp.inf)
    aw = aw_ref[...]
    ah = ah_ref[...]

    pcx = d_ref[:, :, 0, :] * aw + acx_ref[...]
    pcy = d_ref[:, :, 1, :] * ah + acy_ref[...]
    pw = jnp.exp(d_ref[:, :, 2, :]) * aw
    ph = jnp.exp(d_ref[:, :, 3, :]) * ah
    x1 = pcx - 0.5 * pw
    y1 = pcy - 0.5 * ph
    x2 = pcx + 0.5 * pw
    y2 = pcy + 0.5 * ph
    xmax = (im_ref[:, 1:2] - 1.0)[:, :, None]
    ymax = (im_ref[:, 0:1] - 1.0)[:, :, None]
    x1 = jnp.minimum(jnp.maximum(x1, 0.0), xmax)
    x2 = jnp.minimum(jnp.maximum(x2, 0.0), xmax)
    y1 = jnp.minimum(jnp.maximum(y1, 0.0), ymax)
    y2 = jnp.minimum(jnp.maximum(y2, 0.0), ymax)

    sc = sc_ref[:, _A:, :]
    u = lax.bitcast_convert_type(sc, jnp.int32)
    imin = jnp.int32(-2147483648)
    keys = jnp.where(u >= 0, u, imin - u)

    def bs_body(_, lohi):
        lo, hi = lohi
        mid = (lo >> 1) + (hi >> 1) + (lo & hi & 1)
        cnt = jnp.sum(
            jnp.sum((keys >= mid).astype(jnp.int32), axis=2, keepdims=True),
            axis=1, keepdims=True)
        ge = cnt >= _PRE_N
        return jnp.where(ge, mid, lo), jnp.where(ge, hi, mid)

    lo0 = jnp.full((_B, 1, 1), imin, jnp.int32)
    hi0 = jnp.full((_B, 1, 1), 2147483647, jnp.int32)
    tkey, _ = lax.fori_loop(0, 32, bs_body, (lo0, hi0))
    scw = jnp.where(keys >= tkey, sc, NEG)

    fill = jnp.zeros((_B, _A, _HWP - _HW), jnp.float32)
    ox1_ref[:, :, :_HW] = x1
    oy1_ref[:, :, :_HW] = y1
    ox2_ref[:, :, :_HW] = x2
    oy2_ref[:, :, :_HW] = y2
    oar_ref[:, :, :_HW] = (x2 - x1 + 1.0) * (y2 - y1 + 1.0)
    osw_ref[:, :, :_HW] = scw
    ox1_ref[:, :, _HW:] = fill
    oy1_ref[:, :, _HW:] = fill
    ox2_ref[:, :, _HW:] = fill
    oy2_ref[:, :, _HW:] = fill
    oar_ref[:, :, _HW:] = fill
    osw_ref[:, :, _HW:] = fill + NEG


def _sc_body(*refs):
    hin = refs[0:6]
    hout = refs[6:13]
    cina = refs[13:19]
    cinb = refs[19:25]
    cout = refs[25:31]
    coi = refs[31]
    sem = refs[32]
    c = lax.axis_index("c")
    s = lax.axis_index("s")
    img = s * 2 + c
    NEG = jnp.float32(-jnp.inf)
    lane = lax.iota(jnp.int32, 16)

    @pl.when(img < _B)
    def _():
        bufs = [cina, cinb]
        handles = [
            pltpu.async_copy(hr.at[pl.ds(img * _NPAD, _CHUNK)], vr, sem)
            for hr, vr in zip(hin, cina)
        ]

        def init_body(i, carry):
            sl = pl.ds(i * 16, 16)
            for r in cout[:5]:
                r[sl] = jnp.zeros((16,), jnp.float32)
            cout[5][sl] = jnp.full((16,), NEG, jnp.float32)
            coi[sl] = jnp.zeros((16,), jnp.int32)
            return carry
        lax.fori_loop(0, _CAP // 16, init_body, 0)

        wp = jnp.int32(0)
        for ci in range(_NCHUNKS):
            cur = bufs[ci % 2]
            for h in handles:
                h.wait()
            if ci + 1 < _NCHUNKS:
                nxt = bufs[(ci + 1) % 2]
                base = img * _NPAD + (ci + 1) * _CHUNK
                handles = [
                    pltpu.async_copy(hr.at[pl.ds(base, _CHUNK)], vr, sem)
                    for hr, vr in zip(hin, nxt)
                ]

            def vec_body(i, wpc, cur=cur, ci=ci):
                sl = pl.ds(i * 16, 16)
                mask = cur[5][sl] > NEG
                cnt = jnp.sum(mask.astype(jnp.int32))
                p = ci * _CHUNK + i * 16 + lane
                a = p // _HWP
                hw = p - a * _HWP
                oi = hw * _A + a
                mask = jnp.logical_and(mask, wpc <= _CAP - 16)
                wsafe = jnp.minimum(wpc, _CAP - 16)
                for vr, orr in zip(cur, cout):
                    plsc.store_compressed(orr.at[pl.ds(wsafe, 16)],
                                          vr[sl], mask=mask)
                plsc.store_compressed(coi.at[pl.ds(wsafe, 16)], oi, mask=mask)
                return wpc + cnt

            wp = lax.fori_loop(0, _CHUNK // 16, vec_body, wp)

        outh = [
            pltpu.async_copy(orr, hr.at[pl.ds(img * _CAP, _CAP)], sem)
            for orr, hr in zip(list(cout) + [coi], hout)
        ]
        for h in outh:
            h.wait()


@functools.lru_cache(maxsize=1)
def _get_sc_compact():
    return pl.kernel(
        _sc_body,
        out_type=[jax.ShapeDtypeStruct((_B * _CAP,), jnp.float32)] * 6
        + [jax.ShapeDtypeStruct((_B * _CAP,), jnp.int32)],
        mesh=plsc.VectorSubcoreMesh(core_axis_name="c", subcore_axis_name="s"),
        scratch_types=[pltpu.VMEM((_CHUNK,), jnp.float32)] * 12
        + [pltpu.VMEM((_CAP,), jnp.float32)] * 6
        + [pltpu.VMEM((_CAP,), jnp.int32)]
        + [pltpu.SemaphoreType.DMA],
        compiler_params=pltpu.CompilerParams(needs_layout_passes=False),
    )


def _tc2_body(x1_ref, y1_ref, x2_ref, y2_ref, ar_ref, sw_ref, oi_ref,
              ox1_ref, oy1_ref, ox2_ref, oy2_ref, scw_s):
    NEG = jnp.float32(-jnp.inf)
    scw_s[...] = sw_ref[...]
    oi = oi_ref[...]

    def pick(scw):
        m = jnp.max(scw, axis=1, keepdims=True)
        pidx = jnp.min(jnp.where(scw == m, oi, _BIGIDX), axis=1,
                       keepdims=True)
        oh = oi == pidx

        def g(ref):
            return jnp.sum(jnp.where(oh, ref[...], 0.0), axis=1, keepdims=True)

        return m, oh, g(x1_ref), g(y1_ref), g(x2_ref), g(y2_ref)

    _, _, bx1, by1, bx2, by2 = pick(scw_s[...])

    lane_out = lax.broadcasted_iota(jnp.int32, (_B, _POST_N), 1)

    def body(t, _):
        scw = scw_s[...]
        m, oh, sx1, sy1, sx2, sy2 = pick(scw)
        sar = (sx2 - sx1 + 1.0) * (sy2 - sy1 + 1.0)
        ex = m == NEG
        sel = lane_out == t
        ox1_ref[...] = jnp.where(sel, jnp.where(ex, bx1, sx1), ox1_ref[...])
        oy1_ref[...] = jnp.where(sel, jnp.where(ex, by1, sy1), oy1_ref[...])
        ox2_ref[...] = jnp.where(sel, jnp.where(ex, bx2, sx2), ox2_ref[...])
        oy2_ref[...] = jnp.where(sel, jnp.where(ex, by2, sy2), oy2_ref[...])
        xx1 = jnp.maximum(sx1, x1_ref[...])
        yy1 = jnp.maximum(sy1, y1_ref[...])
        xx2 = jnp.minimum(sx2, x2_ref[...])
        yy2 = jnp.minimum(sy2, y2_ref[...])
        w = jnp.maximum(0.0, xx2 - xx1 + 1.0)
        h = jnp.maximum(0.0, yy2 - yy1 + 1.0)
        inter = w * h
        iou = inter / (sar + ar_ref[...] - inter)
        supp = (iou > _IOU_T) | oh
        scw_s[...] = jnp.where(supp, NEG, scw)
        return 0

    lax.fori_loop(0, _POST_N, body, 0)


def kernel(scores, bbox_deltas, im_info, cfg_key):
    sc_raw = scores.reshape(_B, 2 * _A, _HW)
    d_raw = bbox_deltas.reshape(_B, _A, 4, _HW)

    staged = pl.pallas_call(
        _tc1_body,
        out_shape=[jax.ShapeDtypeStruct((_B, _A, _HWP), jnp.float32)] * 6,
        interpret=_INTERPRET,
    )(sc_raw, d_raw,
      jnp.asarray(_AW), jnp.asarray(_AH), jnp.asarray(_ACX), jnp.asarray(_ACY),
      im_info)

    compacted = _get_sc_compact()(*[a.reshape(_B * _NPAD) for a in staged])
    compacted = [a.reshape(_B, _CAP) for a in compacted]

    outs = pl.pallas_call(
        _tc2_body,
        out_shape=[jax.ShapeDtypeStruct((_B, _POST_N), jnp.float32)] * 4,
        scratch_shapes=[pltpu.VMEM((_B, _CAP), jnp.float32)],
        interpret=_INTERPRET,
    )(*compacted)

    x1, y1, x2, y2 = outs
    boxes = jnp.stack([x1, y1, x2, y2], axis=-1)
    bid = jnp.broadcast_to(
        jnp.arange(_B, dtype=jnp.float32)[:, None, None], (_B, _POST_N, 1))
    return jnp.concatenate([bid, boxes], axis=2)


# speedup vs baseline: 5.1235x; 1.0010x over previous
import functools

import numpy as np
import jax
import jax.numpy as jnp
from jax import lax
from jax.experimental import pallas as pl
from jax.experimental.pallas import tpu as pltpu
from jax.experimental.pallas import tpu_sc as plsc

_FEAT_STRIDE = 16
_PRE_N = 6000
_POST_N = 300
_IOU_T = 0.7
_B, _H, _W, _A = 8, 50, 76, 9
_HW = _H * _W
_HWP = 3840
_N = _HW * _A
_NPAD = _HWP * _A
_CHUNK = _NPAD // 8
_NCHUNKS = 8
_CAP = 6144
_BIGIDX = 1 << 28

_INTERPRET = False


def _anchor_params():
    ratios = np.array([0.5, 1.0, 2.0])
    scales = np.array([8.0, 16.0, 32.0])

    def whctrs(a):
        w = a[2] - a[0] + 1.0
        h = a[3] - a[1] + 1.0
        return w, h, a[0] + 0.5 * (w - 1.0), a[1] + 0.5 * (h - 1.0)

    def mk(ws, hs, xc, yc):
        ws = ws[:, None]
        hs = hs[:, None]
        return np.hstack([
            xc - 0.5 * (ws - 1.0), yc - 0.5 * (hs - 1.0),
            xc + 0.5 * (ws - 1.0), yc + 0.5 * (hs - 1.0),
        ])

    base = np.array([0.0, 0.0, 15.0, 15.0])
    w, h, xc, yc = whctrs(base)
    size_ratios = (w * h) / ratios
    ws = np.round(np.sqrt(size_ratios))
    hs = np.round(ws * ratios)
    ra = mk(ws, hs, xc, yc)
    rows = []
    for i in range(ra.shape[0]):
        wi, hi, xci, yci = whctrs(ra[i])
        rows.append(mk(wi * scales, hi * scales, xci, yci))
    anchors32 = np.vstack(rows).astype(np.float32)

    shift_x = np.arange(_W, dtype=np.float32) * np.float32(_FEAT_STRIDE)
    shift_y = np.arange(_H, dtype=np.float32) * np.float32(_FEAT_STRIDE)
    sx, sy = np.meshgrid(shift_x, shift_y)
    shifts = np.stack([sx.ravel(), sy.ravel(), sx.ravel(), sy.ravel()],
                      axis=1).astype(np.float32)
    full = (anchors32[None, :, :] + shifts[:, None, :]).reshape(-1, 4)
    aw = (full[:, 2] - full[:, 0]) + np.float32(1.0)
    ah = (full[:, 3] - full[:, 1]) + np.float32(1.0)
    acx = full[:, 0] + np.float32(0.5) * aw
    acy = full[:, 1] + np.float32(0.5) * ah

    def raw(v):
        return np.ascontiguousarray(v.reshape(_HW, _A).T).reshape(1, _A, _HW)

    return raw(aw), raw(ah), raw(acx), raw(acy)


_AW, _AH, _ACX, _ACY = _anchor_params()


def _tc1_body(sc_ref, d_ref, aw_ref, ah_ref, acx_ref, acy_ref, im_ref,
              ox1_ref, oy1_ref, ox2_ref, oy2_ref, oar_ref, osw_ref):
    NEG = jnp.float32(-jnp.inf)
    aw = aw_ref[...]
    ah = ah_ref[...]

    pcx = d_ref[:, :, 0, :] * aw + acx_ref[...]
    pcy = d_ref[:, :, 1, :] * ah + acy_ref[...]
    pw = jnp.exp(d_ref[:, :, 2, :]) * aw
    ph = jnp.exp(d_ref[:, :, 3, :]) * ah
    x1 = pcx - 0.5 * pw
    y1 = pcy - 0.5 * ph
    x2 = pcx + 0.5 * pw
    y2 = pcy + 0.5 * ph
    xmax = (im_ref[:, 1:2] - 1.0)[:, :, None]
    ymax = (im_ref[:, 0:1] - 1.0)[:, :, None]
    x1 = jnp.minimum(jnp.maximum(x1, 0.0), xmax)
    x2 = jnp.minimum(jnp.maximum(x2, 0.0), xmax)
    y1 = jnp.minimum(jnp.maximum(y1, 0.0), ymax)
    y2 = jnp.minimum(jnp.maximum(y2, 0.0), ymax)

    sc = sc_ref[:, _A:, :]
    u = lax.bitcast_convert_type(sc, jnp.int32)
    imin = jnp.int32(-2147483648)
    keys = jnp.where(u >= 0, u, imin - u)

    def bs_body(_, lohi):
        lo, hi = lohi
        mid = (lo >> 1) + (hi >> 1) + (lo & hi & 1)
        cnt = jnp.sum(
            jnp.sum((keys >= mid).astype(jnp.int32), axis=2, keepdims=True),
            axis=1, keepdims=True)
        ge = cnt >= _PRE_N
        return jnp.where(ge, mid, lo), jnp.where(ge, hi, mid)

    lo0 = jnp.full((_B, 1, 1), imin, jnp.int32)
    hi0 = jnp.full((_B, 1, 1), 2147483647, jnp.int32)
    tkey, _ = lax.fori_loop(0, 32, bs_body, (lo0, hi0))
    scw = jnp.where(keys >= tkey, sc, NEG)

    fill = jnp.zeros((_B, _A, _HWP - _HW), jnp.float32)
    ox1_ref[:, :, :_HW] = x1
    oy1_ref[:, :, :_HW] = y1
    ox2_ref[:, :, :_HW] = x2
    oy2_ref[:, :, :_HW] = y2
    oar_ref[:, :, :_HW] = (x2 - x1 + 1.0) * (y2 - y1 + 1.0)
    osw_ref[:, :, :_HW] = scw
    ox1_ref[:, :, _HW:] = fill
    oy1_ref[:, :, _HW:] = fill
    ox2_ref[:, :, _HW:] = fill
    oy2_ref[:, :, _HW:] = fill
    oar_ref[:, :, _HW:] = fill
    osw_ref[:, :, _HW:] = fill + NEG


def _sc_body(*refs):
    hin = refs[0:6]
    hout = refs[6:13]
    cina = refs[13:19]
    cinb = refs[19:25]
    cout = refs[25:31]
    coi = refs[31]
    sem = refs[32]
    c = lax.axis_index("c")
    s = lax.axis_index("s")
    img = s * 2 + c
    NEG = jnp.float32(-jnp.inf)
    lane = lax.iota(jnp.int32, 16)

    @pl.when(img < _B)
    def _():
        bufs = [cina, cinb]
        handles = [
            pltpu.async_copy(hr.at[pl.ds(img * _NPAD, _CHUNK)], vr, sem)
            for hr, vr in zip(hin, cina)
        ]

        def init_body(i, carry):
            sl = pl.ds(i * 16, 16)
            for r in cout[:5]:
                r[sl] = jnp.zeros((16,), jnp.float32)
            cout[5][sl] = jnp.full((16,), NEG, jnp.float32)
            coi[sl] = jnp.zeros((16,), jnp.int32)
            return carry
        lax.fori_loop(0, _CAP // 16, init_body, 0)

        wp = jnp.int32(0)
        for ci in range(_NCHUNKS):
            cur = bufs[ci % 2]
            for h in handles:
                h.wait()
            if ci + 1 < _NCHUNKS:
                nxt = bufs[(ci + 1) % 2]
                base = img * _NPAD + (ci + 1) * _CHUNK
                handles = [
                    pltpu.async_copy(hr.at[pl.ds(base, _CHUNK)], vr, sem)
                    for hr, vr in zip(hin, nxt)
                ]

            def vec_body(i, wpc, cur=cur, ci=ci):
                sl = pl.ds(i * 16, 16)
                mask = cur[5][sl] > NEG
                cnt = jnp.sum(mask.astype(jnp.int32))
                p = ci * _CHUNK + i * 16 + lane
                a = p // _HWP
                hw = p - a * _HWP
                oi = hw * _A + a
                mask = jnp.logical_and(mask, wpc <= _CAP - 16)
                wsafe = jnp.minimum(wpc, _CAP - 16)
                for vr, orr in zip(cur, cout):
                    plsc.store_compressed(orr.at[pl.ds(wsafe, 16)],
                                          vr[sl], mask=mask)
                plsc.store_compressed(coi.at[pl.ds(wsafe, 16)], oi, mask=mask)
                return wpc + cnt

            wp = lax.fori_loop(0, _CHUNK // 16, vec_body, wp)

        outh = [
            pltpu.async_copy(orr, hr.at[pl.ds(img * _CAP, _CAP)], sem)
            for orr, hr in zip(list(cout) + [coi], hout)
        ]
        for h in outh:
            h.wait()


@functools.lru_cache(maxsize=1)
def _get_sc_compact():
    return pl.kernel(
        _sc_body,
        out_type=[jax.ShapeDtypeStruct((_B * _CAP,), jnp.float32)] * 6
        + [jax.ShapeDtypeStruct((_B * _CAP,), jnp.int32)],
        mesh=plsc.VectorSubcoreMesh(core_axis_name="c", subcore_axis_name="s"),
        scratch_types=[pltpu.VMEM((_CHUNK,), jnp.float32)] * 12
        + [pltpu.VMEM((_CAP,), jnp.float32)] * 6
        + [pltpu.VMEM((_CAP,), jnp.int32)]
        + [pltpu.SemaphoreType.DMA],
        compiler_params=pltpu.CompilerParams(needs_layout_passes=False),
    )


def _tc2_body(x1_ref, y1_ref, x2_ref, y2_ref, ar_ref, sw_ref, oi_ref,
              ox1_ref, oy1_ref, ox2_ref, oy2_ref, scw_s):
    NEG = jnp.float32(-jnp.inf)
    scw_s[...] = sw_ref[...]
    oi = oi_ref[...]

    def pick(scw):
        m = jnp.max(scw, axis=1, keepdims=True)
        pidx = jnp.min(jnp.where(scw == m, oi, _BIGIDX), axis=1,
                       keepdims=True)
        oh = oi == pidx
        stacked = jnp.concatenate(
            [jnp.where(oh, x1_ref[...], 0.0),
             jnp.where(oh, y1_ref[...], 0.0),
             jnp.where(oh, x2_ref[...], 0.0),
             jnp.where(oh, y2_ref[...], 0.0)], axis=0)
        g = jnp.sum(stacked, axis=1, keepdims=True)
        return (m, oh, g[0:_B], g[_B:2 * _B],
                g[2 * _B:3 * _B], g[3 * _B:4 * _B])

    _, _, bx1, by1, bx2, by2 = pick(scw_s[...])

    lane_out = lax.broadcasted_iota(jnp.int32, (_B, _POST_N), 1)

    def body(t, _):
        scw = scw_s[...]
        m, oh, sx1, sy1, sx2, sy2 = pick(scw)
        sar = (sx2 - sx1 + 1.0) * (sy2 - sy1 + 1.0)
        ex = m == NEG
        sel = lane_out == t
        ox1_ref[...] = jnp.where(sel, jnp.where(ex, bx1, sx1), ox1_ref[...])
        oy1_ref[...] = jnp.where(sel, jnp.where(ex, by1, sy1), oy1_ref[...])
        ox2_ref[...] = jnp.where(sel, jnp.where(ex, bx2, sx2), ox2_ref[...])
        oy2_ref[...] = jnp.where(sel, jnp.where(ex, by2, sy2), oy2_ref[...])
        xx1 = jnp.maximum(sx1, x1_ref[...])
        yy1 = jnp.maximum(sy1, y1_ref[...])
        xx2 = jnp.minimum(sx2, x2_ref[...])
        yy2 = jnp.minimum(sy2, y2_ref[...])
        w = jnp.maximum(0.0, xx2 - xx1 + 1.0)
        h = jnp.maximum(0.0, yy2 - yy1 + 1.0)
        inter = w * h
        iou = inter / (sar + ar_ref[...] - inter)
        supp = (iou > _IOU_T) | oh
        scw_s[...] = jnp.where(supp, NEG, scw)
        return 0

    lax.fori_loop(0, _POST_N, body, 0)


def kernel(scores, bbox_deltas, im_info, cfg_key):
    sc_raw = scores.reshape(_B, 2 * _A, _HW)
    d_raw = bbox_deltas.reshape(_B, _A, 4, _HW)

    staged = pl.pallas_call(
        _tc1_body,
        out_shape=[jax.ShapeDtypeStruct((_B, _A, _HWP), jnp.float32)] * 6,
        interpret=_INTERPRET,
    )(sc_raw, d_raw,
      jnp.asarray(_AW), jnp.asarray(_AH), jnp.asarray(_ACX), jnp.asarray(_ACY),
      im_info)

    compacted = _get_sc_compact()(*[a.reshape(_B * _NPAD) for a in staged])
    compacted = [a.reshape(_B, _CAP) for a in compacted]

    outs = pl.pallas_call(
        _tc2_body,
        out_shape=[jax.ShapeDtypeStruct((_B, _POST_N), jnp.float32)] * 4,
        scratch_shapes=[pltpu.VMEM((_B, _CAP), jnp.float32)],
        interpret=_INTERPRET,
    )(*compacted)

    x1, y1, x2, y2 = outs
    boxes = jnp.stack([x1, y1, x2, y2], axis=-1)
    bid = jnp.broadcast_to(
        jnp.arange(_B, dtype=jnp.float32)[:, None, None], (_B, _POST_N, 1))
    return jnp.concatenate([bid, boxes], axis=2)


# speedup vs baseline: 5.7423x; 1.1208x over previous
import functools

import numpy as np
import jax
import jax.numpy as jnp
from jax import lax
from jax.experimental import pallas as pl
from jax.experimental.pallas import tpu as pltpu
from jax.experimental.pallas import tpu_sc as plsc

_FEAT_STRIDE = 16
_PRE_N = 6000
_POST_N = 300
_IOU_T = 0.7
_B, _H, _W, _A = 8, 50, 76, 9
_HW = _H * _W
_HWP = 3840
_N = _HW * _A
_NPAD = _HWP * _A
_CHUNK = _NPAD // 8
_NCHUNKS = 8
_CAP = 6144
_BIGIDX = 1 << 28

_INTERPRET = False


def _anchor_params():
    ratios = np.array([0.5, 1.0, 2.0])
    scales = np.array([8.0, 16.0, 32.0])

    def whctrs(a):
        w = a[2] - a[0] + 1.0
        h = a[3] - a[1] + 1.0
        return w, h, a[0] + 0.5 * (w - 1.0), a[1] + 0.5 * (h - 1.0)

    def mk(ws, hs, xc, yc):
        ws = ws[:, None]
        hs = hs[:, None]
        return np.hstack([
            xc - 0.5 * (ws - 1.0), yc - 0.5 * (hs - 1.0),
            xc + 0.5 * (ws - 1.0), yc + 0.5 * (hs - 1.0),
        ])

    base = np.array([0.0, 0.0, 15.0, 15.0])
    w, h, xc, yc = whctrs(base)
    size_ratios = (w * h) / ratios
    ws = np.round(np.sqrt(size_ratios))
    hs = np.round(ws * ratios)
    ra = mk(ws, hs, xc, yc)
    rows = []
    for i in range(ra.shape[0]):
        wi, hi, xci, yci = whctrs(ra[i])
        rows.append(mk(wi * scales, hi * scales, xci, yci))
    anchors32 = np.vstack(rows).astype(np.float32)

    shift_x = np.arange(_W, dtype=np.float32) * np.float32(_FEAT_STRIDE)
    shift_y = np.arange(_H, dtype=np.float32) * np.float32(_FEAT_STRIDE)
    sx, sy = np.meshgrid(shift_x, shift_y)
    shifts = np.stack([sx.ravel(), sy.ravel(), sx.ravel(), sy.ravel()],
                      axis=1).astype(np.float32)
    full = (anchors32[None, :, :] + shifts[:, None, :]).reshape(-1, 4)
    aw = (full[:, 2] - full[:, 0]) + np.float32(1.0)
    ah = (full[:, 3] - full[:, 1]) + np.float32(1.0)
    acx = full[:, 0] + np.float32(0.5) * aw
    acy = full[:, 1] + np.float32(0.5) * ah

    def raw(v):
        return np.ascontiguousarray(v.reshape(_HW, _A).T).reshape(1, _A, _HW)

    return raw(aw), raw(ah), raw(acx), raw(acy)


_AW, _AH, _ACX, _ACY = _anchor_params()


def _tc1_body(sc_ref, d_ref, aw_ref, ah_ref, acx_ref, acy_ref, im_ref,
              ox1_ref, oy1_ref, ox2_ref, oy2_ref, oar_ref, osw_ref):
    NEG = jnp.float32(-jnp.inf)
    aw = aw_ref[...]
    ah = ah_ref[...]

    pcx = d_ref[:, :, 0, :] * aw + acx_ref[...]
    pcy = d_ref[:, :, 1, :] * ah + acy_ref[...]
    pw = jnp.exp(d_ref[:, :, 2, :]) * aw
    ph = jnp.exp(d_ref[:, :, 3, :]) * ah
    x1 = pcx - 0.5 * pw
    y1 = pcy - 0.5 * ph
    x2 = pcx + 0.5 * pw
    y2 = pcy + 0.5 * ph
    xmax = (im_ref[:, 1:2] - 1.0)[:, :, None]
    ymax = (im_ref[:, 0:1] - 1.0)[:, :, None]
    x1 = jnp.minimum(jnp.maximum(x1, 0.0), xmax)
    x2 = jnp.minimum(jnp.maximum(x2, 0.0), xmax)
    y1 = jnp.minimum(jnp.maximum(y1, 0.0), ymax)
    y2 = jnp.minimum(jnp.maximum(y2, 0.0), ymax)

    sc = sc_ref[:, _A:, :]
    u = lax.bitcast_convert_type(sc, jnp.int32)
    imin = jnp.int32(-2147483648)
    keys = jnp.where(u >= 0, u, imin - u)

    def bs_body(_, lohi):
        lo, hi = lohi
        mid = (lo >> 1) + (hi >> 1) + (lo & hi & 1)
        cnt = jnp.sum(
            jnp.sum((keys >= mid).astype(jnp.int32), axis=2, keepdims=True),
            axis=1, keepdims=True)
        ge = cnt >= _PRE_N
        return jnp.where(ge, mid, lo), jnp.where(ge, hi, mid)

    lo0 = jnp.full((_B, 1, 1), imin, jnp.int32)
    hi0 = jnp.full((_B, 1, 1), 2147483647, jnp.int32)
    tkey, _ = lax.fori_loop(0, 32, bs_body, (lo0, hi0))
    scw = jnp.where(keys >= tkey, sc, NEG)

    fill = jnp.zeros((_B, _A, _HWP - _HW), jnp.float32)
    ox1_ref[:, :, :_HW] = x1
    oy1_ref[:, :, :_HW] = y1
    ox2_ref[:, :, :_HW] = x2
    oy2_ref[:, :, :_HW] = y2
    oar_ref[:, :, :_HW] = (x2 - x1 + 1.0) * (y2 - y1 + 1.0)
    osw_ref[:, :, :_HW] = scw
    ox1_ref[:, :, _HW:] = fill
    oy1_ref[:, :, _HW:] = fill
    ox2_ref[:, :, _HW:] = fill
    oy2_ref[:, :, _HW:] = fill
    oar_ref[:, :, _HW:] = fill
    osw_ref[:, :, _HW:] = fill + NEG


def _sc_body(*refs):
    hin = refs[0:6]
    hout = refs[6:13]
    cina = refs[13:19]
    cinb = refs[19:25]
    cout = refs[25:31]
    coi = refs[31]
    sem = refs[32]
    c = lax.axis_index("c")
    s = lax.axis_index("s")
    img = s * 2 + c
    NEG = jnp.float32(-jnp.inf)
    lane = lax.iota(jnp.int32, 16)

    @pl.when(img < _B)
    def _():
        bufs = [cina, cinb]
        handles = [
            pltpu.async_copy(hr.at[pl.ds(img * _NPAD, _CHUNK)], vr, sem)
            for hr, vr in zip(hin, cina)
        ]

        def init_body(i, carry):
            sl = pl.ds(i * 16, 16)
            for r in cout[:5]:
                r[sl] = jnp.zeros((16,), jnp.float32)
            cout[5][sl] = jnp.full((16,), NEG, jnp.float32)
            coi[sl] = jnp.zeros((16,), jnp.int32)
            return carry
        lax.fori_loop(0, _CAP // 16, init_body, 0)

        wp = jnp.int32(0)
        for ci in range(_NCHUNKS):
            cur = bufs[ci % 2]
            for h in handles:
                h.wait()
            if ci + 1 < _NCHUNKS:
                nxt = bufs[(ci + 1) % 2]
                base = img * _NPAD + (ci + 1) * _CHUNK
                handles = [
                    pltpu.async_copy(hr.at[pl.ds(base, _CHUNK)], vr, sem)
                    for hr, vr in zip(hin, nxt)
                ]

            @plsc.parallel_loop(0, _CHUNK // 16, unroll=4, carry=wp)
            def wp(i, wpc, cur=cur, ci=ci):
                sl = pl.ds(i * 16, 16)
                mask = cur[5][sl] > NEG
                cnt = jnp.sum(mask.astype(jnp.int32))
                p = ci * _CHUNK + i * 16 + lane
                a = p // _HWP
                hw = p - a * _HWP
                oi = hw * _A + a
                mask = jnp.logical_and(mask, wpc <= _CAP - 16)
                wsafe = jnp.minimum(wpc, _CAP - 16)
                for vr, orr in zip(cur, cout):
                    plsc.store_compressed(orr.at[pl.ds(wsafe, 16)],
                                          vr[sl], mask=mask)
                plsc.store_compressed(coi.at[pl.ds(wsafe, 16)], oi, mask=mask)
                return wpc + cnt

        outh = [
            pltpu.async_copy(orr, hr.at[pl.ds(img * _CAP, _CAP)], sem)
            for orr, hr in zip(list(cout) + [coi], hout)
        ]
        for h in outh:
            h.wait()


@functools.lru_cache(maxsize=1)
def _get_sc_compact():
    return pl.kernel(
        _sc_body,
        out_type=[jax.ShapeDtypeStruct((_B * _CAP,), jnp.float32)] * 6
        + [jax.ShapeDtypeStruct((_B * _CAP,), jnp.int32)],
        mesh=plsc.VectorSubcoreMesh(core_axis_name="c", subcore_axis_name="s"),
        scratch_types=[pltpu.VMEM((_CHUNK,), jnp.float32)] * 12
        + [pltpu.VMEM((_CAP,), jnp.float32)] * 6
        + [pltpu.VMEM((_CAP,), jnp.int32)]
        + [pltpu.SemaphoreType.DMA],
        compiler_params=pltpu.CompilerParams(needs_layout_passes=False),
    )


def _tc2_body(x1_ref, y1_ref, x2_ref, y2_ref, ar_ref, sw_ref, oi_ref,
              ox1_ref, oy1_ref, ox2_ref, oy2_ref, scw_s):
    NEG = jnp.float32(-jnp.inf)
    scw_s[...] = sw_ref[...]
    oi = oi_ref[...]

    def pick(scw):
        m = jnp.max(scw, axis=1, keepdims=True)
        pidx = jnp.min(jnp.where(scw == m, oi, _BIGIDX), axis=1,
                       keepdims=True)
        oh = oi == pidx
        stacked = jnp.concatenate(
            [jnp.where(oh, x1_ref[...], 0.0),
             jnp.where(oh, y1_ref[...], 0.0),
             jnp.where(oh, x2_ref[...], 0.0),
             jnp.where(oh, y2_ref[...], 0.0)], axis=0)
        g = jnp.sum(stacked, axis=1, keepdims=True)
        return (m, oh, g[0:_B], g[_B:2 * _B],
                g[2 * _B:3 * _B], g[3 * _B:4 * _B])

    _, _, bx1, by1, bx2, by2 = pick(scw_s[...])

    lane_out = lax.broadcasted_iota(jnp.int32, (_B, _POST_N), 1)

    def body(t, _):
        scw = scw_s[...]
        m, oh, sx1, sy1, sx2, sy2 = pick(scw)
        sar = (sx2 - sx1 + 1.0) * (sy2 - sy1 + 1.0)
        ex = m == NEG
        sel = lane_out == t
        ox1_ref[...] = jnp.where(sel, jnp.where(ex, bx1, sx1), ox1_ref[...])
        oy1_ref[...] = jnp.where(sel, jnp.where(ex, by1, sy1), oy1_ref[...])
        ox2_ref[...] = jnp.where(sel, jnp.where(ex, bx2, sx2), ox2_ref[...])
        oy2_ref[...] = jnp.where(sel, jnp.where(ex, by2, sy2), oy2_ref[...])
        xx1 = jnp.maximum(sx1, x1_ref[...])
        yy1 = jnp.maximum(sy1, y1_ref[...])
        xx2 = jnp.minimum(sx2, x2_ref[...])
        yy2 = jnp.minimum(sy2, y2_ref[...])
        w = jnp.maximum(0.0, xx2 - xx1 + 1.0)
        h = jnp.maximum(0.0, yy2 - yy1 + 1.0)
        inter = w * h
        iou = inter / (sar + ar_ref[...] - inter)
        supp = (iou > _IOU_T) | oh
        scw_s[...] = jnp.where(supp, NEG, scw)
        return 0

    lax.fori_loop(0, _POST_N, body, 0)


def kernel(scores, bbox_deltas, im_info, cfg_key):
    sc_raw = scores.reshape(_B, 2 * _A, _HW)
    d_raw = bbox_deltas.reshape(_B, _A, 4, _HW)

    staged = pl.pallas_call(
        _tc1_body,
        out_shape=[jax.ShapeDtypeStruct((_B, _A, _HWP), jnp.float32)] * 6,
        interpret=_INTERPRET,
    )(sc_raw, d_raw,
      jnp.asarray(_AW), jnp.asarray(_AH), jnp.asarray(_ACX), jnp.asarray(_ACY),
      im_info)

    compacted = _get_sc_compact()(*[a.reshape(_B * _NPAD) for a in staged])
    compacted = [a.reshape(_B, _CAP) for a in compacted]

    outs = pl.pallas_call(
        _tc2_body,
        out_shape=[jax.ShapeDtypeStruct((_B, _POST_N), jnp.float32)] * 4,
        scratch_shapes=[pltpu.VMEM((_B, _CAP), jnp.float32)],
        interpret=_INTERPRET,
    )(*compacted)

    x1, y1, x2, y2 = outs
    boxes = jnp.stack([x1, y1, x2, y2], axis=-1)
    bid = jnp.broadcast_to(
        jnp.arange(_B, dtype=jnp.float32)[:, None, None], (_B, _POST_N, 1))
    return jnp.concatenate([bid, boxes], axis=2)


# speedup vs baseline: 5.8036x; 1.0107x over previous
import functools

import numpy as np
import jax
import jax.numpy as jnp
from jax import lax
from jax.experimental import pallas as pl
from jax.experimental.pallas import tpu as pltpu
from jax.experimental.pallas import tpu_sc as plsc

_FEAT_STRIDE = 16
_PRE_N = 6000
_POST_N = 300
_IOU_T = 0.7
_B, _H, _W, _A = 8, 50, 76, 9
_HW = _H * _W
_HWP = 3840
_N = _HW * _A
_NPAD = _HWP * _A
_CHUNK = _NPAD // 8
_NCHUNKS = 8
_CAP = 6144
_BIGIDX = 1 << 28

_INTERPRET = False


def _anchor_params():
    ratios = np.array([0.5, 1.0, 2.0])
    scales = np.array([8.0, 16.0, 32.0])

    def whctrs(a):
        w = a[2] - a[0] + 1.0
        h = a[3] - a[1] + 1.0
        return w, h, a[0] + 0.5 * (w - 1.0), a[1] + 0.5 * (h - 1.0)

    def mk(ws, hs, xc, yc):
        ws = ws[:, None]
        hs = hs[:, None]
        return np.hstack([
            xc - 0.5 * (ws - 1.0), yc - 0.5 * (hs - 1.0),
            xc + 0.5 * (ws - 1.0), yc + 0.5 * (hs - 1.0),
        ])

    base = np.array([0.0, 0.0, 15.0, 15.0])
    w, h, xc, yc = whctrs(base)
    size_ratios = (w * h) / ratios
    ws = np.round(np.sqrt(size_ratios))
    hs = np.round(ws * ratios)
    ra = mk(ws, hs, xc, yc)
    rows = []
    for i in range(ra.shape[0]):
        wi, hi, xci, yci = whctrs(ra[i])
        rows.append(mk(wi * scales, hi * scales, xci, yci))
    anchors32 = np.vstack(rows).astype(np.float32)

    shift_x = np.arange(_W, dtype=np.float32) * np.float32(_FEAT_STRIDE)
    shift_y = np.arange(_H, dtype=np.float32) * np.float32(_FEAT_STRIDE)
    sx, sy = np.meshgrid(shift_x, shift_y)
    shifts = np.stack([sx.ravel(), sy.ravel(), sx.ravel(), sy.ravel()],
                      axis=1).astype(np.float32)
    full = (anchors32[None, :, :] + shifts[:, None, :]).reshape(-1, 4)
    aw = (full[:, 2] - full[:, 0]) + np.float32(1.0)
    ah = (full[:, 3] - full[:, 1]) + np.float32(1.0)
    acx = full[:, 0] + np.float32(0.5) * aw
    acy = full[:, 1] + np.float32(0.5) * ah

    def raw(v):
        return np.ascontiguousarray(v.reshape(_HW, _A).T).reshape(1, _A, _HW)

    return raw(aw), raw(ah), raw(acx), raw(acy)


_AW, _AH, _ACX, _ACY = _anchor_params()


def _tc1_body(sc_ref, d_ref, aw_ref, ah_ref, acx_ref, acy_ref, im_ref,
              ox1_ref, oy1_ref, ox2_ref, oy2_ref, oar_ref, osw_ref):
    NEG = jnp.float32(-jnp.inf)
    aw = aw_ref[...]
    ah = ah_ref[...]

    pcx = d_ref[:, :, 0, :] * aw + acx_ref[...]
    pcy = d_ref[:, :, 1, :] * ah + acy_ref[...]
    pw = jnp.exp(d_ref[:, :, 2, :]) * aw
    ph = jnp.exp(d_ref[:, :, 3, :]) * ah
    x1 = pcx - 0.5 * pw
    y1 = pcy - 0.5 * ph
    x2 = pcx + 0.5 * pw
    y2 = pcy + 0.5 * ph
    xmax = (im_ref[:, 1:2] - 1.0)[:, :, None]
    ymax = (im_ref[:, 0:1] - 1.0)[:, :, None]
    x1 = jnp.minimum(jnp.maximum(x1, 0.0), xmax)
    x2 = jnp.minimum(jnp.maximum(x2, 0.0), xmax)
    y1 = jnp.minimum(jnp.maximum(y1, 0.0), ymax)
    y2 = jnp.minimum(jnp.maximum(y2, 0.0), ymax)

    sc = sc_ref[:, _A:, :]
    u = lax.bitcast_convert_type(sc, jnp.int32)
    imin = jnp.int32(-2147483648)
    keys = jnp.where(u >= 0, u, imin - u)

    def bs_body(_, lohi):
        lo, hi = lohi
        mid = (lo >> 1) + (hi >> 1) + (lo & hi & 1)
        cnt = jnp.sum(
            jnp.sum((keys >= mid).astype(jnp.int32), axis=2, keepdims=True),
            axis=1, keepdims=True)
        ge = cnt >= _PRE_N
        return jnp.where(ge, mid, lo), jnp.where(ge, hi, mid)

    lo0 = jnp.full((_B, 1, 1), imin, jnp.int32)
    hi0 = jnp.full((_B, 1, 1), 2147483647, jnp.int32)
    tkey, _ = lax.fori_loop(0, 32, bs_body, (lo0, hi0))
    scw = jnp.where(keys >= tkey, sc, NEG)

    fill = jnp.zeros((_B, _A, _HWP - _HW), jnp.float32)
    ox1_ref[:, :, :_HW] = x1
    oy1_ref[:, :, :_HW] = y1
    ox2_ref[:, :, :_HW] = x2
    oy2_ref[:, :, :_HW] = y2
    oar_ref[:, :, :_HW] = (x2 - x1 + 1.0) * (y2 - y1 + 1.0)
    osw_ref[:, :, :_HW] = scw
    ox1_ref[:, :, _HW:] = fill
    oy1_ref[:, :, _HW:] = fill
    ox2_ref[:, :, _HW:] = fill
    oy2_ref[:, :, _HW:] = fill
    oar_ref[:, :, _HW:] = fill
    osw_ref[:, :, _HW:] = fill + NEG


def _sc_body(*refs):
    hin = refs[0:6]
    hout = refs[6:13]
    cina = refs[13:19]
    cinb = refs[19:25]
    cout = refs[25:31]
    coi = refs[31]
    sem = refs[32]
    c = lax.axis_index("c")
    s = lax.axis_index("s")
    img = s * 2 + c
    NEG = jnp.float32(-jnp.inf)
    lane = lax.iota(jnp.int32, 16)

    @pl.when(img < _B)
    def _():
        bufs = [cina, cinb]
        handles = [
            pltpu.async_copy(hr.at[pl.ds(img * _NPAD, _CHUNK)], vr, sem)
            for hr, vr in zip(hin, cina)
        ]

        @plsc.parallel_loop(0, _CAP // 16, unroll=8)
        def _init(i):
            sl = pl.ds(i * 16, 16)
            for r in cout[:5]:
                r[sl] = jnp.zeros((16,), jnp.float32)
            cout[5][sl] = jnp.full((16,), NEG, jnp.float32)
            coi[sl] = jnp.zeros((16,), jnp.int32)

        wp = jnp.int32(0)
        for ci in range(_NCHUNKS):
            cur = bufs[ci % 2]
            for h in handles:
                h.wait()
            if ci + 1 < _NCHUNKS:
                nxt = bufs[(ci + 1) % 2]
                base = img * _NPAD + (ci + 1) * _CHUNK
                handles = [
                    pltpu.async_copy(hr.at[pl.ds(base, _CHUNK)], vr, sem)
                    for hr, vr in zip(hin, nxt)
                ]

            @plsc.parallel_loop(0, _CHUNK // 16, unroll=8, carry=wp)
            def wp(i, wpc, cur=cur, ci=ci):
                sl = pl.ds(i * 16, 16)
                mask = cur[5][sl] > NEG
                cnt = jnp.sum(mask.astype(jnp.int32))
                p = ci * _CHUNK + i * 16 + lane
                a = p // _HWP
                hw = p - a * _HWP
                oi = hw * _A + a
                mask = jnp.logical_and(mask, wpc <= _CAP - 16)
                wsafe = jnp.minimum(wpc, _CAP - 16)
                for vr, orr in zip(cur, cout):
                    plsc.store_compressed(orr.at[pl.ds(wsafe, 16)],
                                          vr[sl], mask=mask)
                plsc.store_compressed(coi.at[pl.ds(wsafe, 16)], oi, mask=mask)
                return wpc + cnt

        outh = [
            pltpu.async_copy(orr, hr.at[pl.ds(img * _CAP, _CAP)], sem)
            for orr, hr in zip(list(cout) + [coi], hout)
        ]
        for h in outh:
            h.wait()


@functools.lru_cache(maxsize=1)
def _get_sc_compact():
    return pl.kernel(
        _sc_body,
        out_type=[jax.ShapeDtypeStruct((_B * _CAP,), jnp.float32)] * 6
        + [jax.ShapeDtypeStruct((_B * _CAP,), jnp.int32)],
        mesh=plsc.VectorSubcoreMesh(core_axis_name="c", subcore_axis_name="s"),
        scratch_types=[pltpu.VMEM((_CHUNK,), jnp.float32)] * 12
        + [pltpu.VMEM((_CAP,), jnp.float32)] * 6
        + [pltpu.VMEM((_CAP,), jnp.int32)]
        + [pltpu.SemaphoreType.DMA],
        compiler_params=pltpu.CompilerParams(needs_layout_passes=False),
    )


def _tc2_body(x1_ref, y1_ref, x2_ref, y2_ref, ar_ref, sw_ref, oi_ref,
              ox1_ref, oy1_ref, ox2_ref, oy2_ref, scw_s):
    NEG = jnp.float32(-jnp.inf)
    scw_s[...] = sw_ref[...]
    oi = oi_ref[...]

    def pick(scw):
        m = jnp.max(scw, axis=1, keepdims=True)
        pidx = jnp.min(jnp.where(scw == m, oi, _BIGIDX), axis=1,
                       keepdims=True)
        oh = oi == pidx
        stacked = jnp.concatenate(
            [jnp.where(oh, x1_ref[...], 0.0),
             jnp.where(oh, y1_ref[...], 0.0),
             jnp.where(oh, x2_ref[...], 0.0),
             jnp.where(oh, y2_ref[...], 0.0)], axis=0)
        g = jnp.sum(stacked, axis=1, keepdims=True)
        return (m, oh, g[0:_B], g[_B:2 * _B],
                g[2 * _B:3 * _B], g[3 * _B:4 * _B])

    _, _, bx1, by1, bx2, by2 = pick(scw_s[...])

    lane_out = lax.broadcasted_iota(jnp.int32, (_B, _POST_N), 1)

    def body(t, _):
        scw = scw_s[...]
        m, oh, sx1, sy1, sx2, sy2 = pick(scw)
        sar = (sx2 - sx1 + 1.0) * (sy2 - sy1 + 1.0)
        ex = m == NEG
        sel = lane_out == t
        ox1_ref[...] = jnp.where(sel, jnp.where(ex, bx1, sx1), ox1_ref[...])
        oy1_ref[...] = jnp.where(sel, jnp.where(ex, by1, sy1), oy1_ref[...])
        ox2_ref[...] = jnp.where(sel, jnp.where(ex, bx2, sx2), ox2_ref[...])
        oy2_ref[...] = jnp.where(sel, jnp.where(ex, by2, sy2), oy2_ref[...])
        xx1 = jnp.maximum(sx1, x1_ref[...])
        yy1 = jnp.maximum(sy1, y1_ref[...])
        xx2 = jnp.minimum(sx2, x2_ref[...])
        yy2 = jnp.minimum(sy2, y2_ref[...])
        w = jnp.maximum(0.0, xx2 - xx1 + 1.0)
        h = jnp.maximum(0.0, yy2 - yy1 + 1.0)
        inter = w * h
        iou = inter / (sar + ar_ref[...] - inter)
        supp = (iou > _IOU_T) | oh
        scw_s[...] = jnp.where(supp, NEG, scw)
        return 0

    lax.fori_loop(0, _POST_N, body, 0)


def kernel(scores, bbox_deltas, im_info, cfg_key):
    sc_raw = scores.reshape(_B, 2 * _A, _HW)
    d_raw = bbox_deltas.reshape(_B, _A, 4, _HW)

    staged = pl.pallas_call(
        _tc1_body,
        out_shape=[jax.ShapeDtypeStruct((_B, _A, _HWP), jnp.float32)] * 6,
        interpret=_INTERPRET,
    )(sc_raw, d_raw,
      jnp.asarray(_AW), jnp.asarray(_AH), jnp.asarray(_ACX), jnp.asarray(_ACY),
      im_info)

    compacted = _get_sc_compact()(*[a.reshape(_B * _NPAD) for a in staged])
    compacted = [a.reshape(_B, _CAP) for a in compacted]

    outs = pl.pallas_call(
        _tc2_body,
        out_shape=[jax.ShapeDtypeStruct((_B, _POST_N), jnp.float32)] * 4,
        scratch_shapes=[pltpu.VMEM((_B, _CAP), jnp.float32)],
        interpret=_INTERPRET,
    )(*compacted)

    x1, y1, x2, y2 = outs
    boxes = jnp.stack([x1, y1, x2, y2], axis=-1)
    bid = jnp.broadcast_to(
        jnp.arange(_B, dtype=jnp.float32)[:, None, None], (_B, _POST_N, 1))
    return jnp.concatenate([bid, boxes], axis=2)
